# baseline (speedup 1.0000x reference)
import functools

import jax
import jax.numpy as jnp
from jax import lax
from jax.experimental import pallas as pl
from jax.experimental.pallas import tpu as pltpu
from jax.experimental.pallas import tpu_sc as plsc

_NC, _NS = 2, 16
_NW = _NC * _NS
_CHUNK = 128
_BPC = 2
_PAD = 56
_NBUF = 5
_D = 3
_WROWS = _PAD * _BPC


def _emb_body(idx_hbm, table_hbm, out_hbm, idx_v, *bufs):
    rows = bufs[:_NBUF]
    gsem = bufs[_NBUF:2 * _NBUF]
    wsem = bufs[2 * _NBUF:3 * _NBUF]

    wid = lax.axis_index("s") * _NC + lax.axis_index("c")
    n = idx_hbm.shape[1]
    wbase = wid * n * _BPC
    pltpu.sync_copy(idx_hbm.at[wid], idx_v)

    def body(j, _):
        @pl.when(j < n)
        def _():
            slot = lax.rem(j, _NBUF)
            for b in range(_NBUF):
                @pl.when(slot == b)
                def _():
                    @pl.when(j >= _NBUF)
                    def _():
                        pltpu.make_async_copy(
                            rows[b].at[pl.ds(0, _WROWS)],
                            out_hbm.at[pl.ds(0, _WROWS)],
                            wsem[b]).wait()
                    pltpu.async_copy(table_hbm.at[idx_v.at[j]],
                                     rows[b], gsem[b])

        i = j - _D
        @pl.when(i >= 0)
        def _():
            slot = lax.rem(i, _NBUF)
            for b in range(_NBUF):
                @pl.when(slot == b)
                def _():
                    pltpu.make_async_copy(table_hbm.at[idx_v.at[i]],
                                          rows[b], gsem[b]).wait()
                    pltpu.async_copy(
                        rows[b].at[pl.ds(0, _WROWS)],
                        out_hbm.at[pl.ds((wbase + i * _BPC) * _PAD, _WROWS)],
                        wsem[b])
        return 0

    lax.fori_loop(0, n + _D, body, 0)

    for b in range(_NBUF):
        pltpu.make_async_copy(
            rows[b].at[pl.ds(0, _WROWS)],
            out_hbm.at[pl.ds(0, _WROWS)],
            wsem[b]).wait()


def kernel(token_ids, lookup):
    bsz, seq = token_ids.shape
    num, dim = lookup.shape
    bpw = bsz // _NW
    n = bpw // _BPC

    t4 = token_ids.astype(jnp.int32).reshape(_NW, n, _BPC, seq)
    t4 = jnp.pad(t4, ((0, 0), (0, 0), (0, 0), (0, _PAD - seq)))
    idx = jnp.pad(t4.reshape(_NW, n, _BPC * _PAD),
                  ((0, 0), (0, 0), (0, _CHUNK - _BPC * _PAD)))

    call = functools.partial(
        pl.kernel,
        mesh=plsc.VectorSubcoreMesh(core_axis_name="c", subcore_axis_name="s"),
        out_type=jax.ShapeDtypeStruct((bsz * _PAD, dim), jnp.float32),
        scratch_types=(
            [pltpu.VMEM((n, _CHUNK), jnp.int32)]
            + [pltpu.VMEM((_CHUNK, dim), jnp.float32) for _ in range(_NBUF)]
            + [pltpu.SemaphoreType.DMA for _ in range(2 * _NBUF)]
        ),
    )(_emb_body)

    out = call(idx, lookup)
    return out.reshape(bsz, _PAD, dim)[:, :seq, :]


# speedup vs baseline: 12.4523x; 12.4523x over previous
import functools

import jax
import jax.numpy as jnp
from jax import lax
from jax.experimental import pallas as pl
from jax.experimental.pallas import tpu as pltpu
from jax.experimental.pallas import tpu_sc as plsc

_NC, _NS = 2, 16
_NW = _NC * _NS
_CHUNK = 128
_BPC = 2
_PAD = 56
_NBUF = 5
_D = 3
_WROWS = _PAD * _BPC


def _emb_body(idx_hbm, table_hbm, out_hbm, idx_v, *bufs):
    rows = bufs[:_NBUF]
    gsem = bufs[_NBUF:2 * _NBUF]
    wsem = bufs[2 * _NBUF:3 * _NBUF]

    wid = lax.axis_index("s") * _NC + lax.axis_index("c")
    n = idx_hbm.shape[1]
    wbase = wid * n * _BPC
    pltpu.sync_copy(idx_hbm.at[wid], idx_v)

    def body(j, _):
        @pl.when(j < n)
        def _():
            slot = lax.rem(j, _NBUF)
            for b in range(_NBUF):
                @pl.when(slot == b)
                def _():
                    @pl.when(j >= _NBUF)
                    def _():
                        pltpu.make_async_copy(
                            rows[b].at[pl.ds(0, _WROWS)],
                            out_hbm.at[pl.ds(0, _WROWS)],
                            wsem[b]).wait()
                    pltpu.async_copy(table_hbm.at[idx_v.at[j]],
                                     rows[b], gsem[b])

        i = j - _D
        @pl.when(i >= 0)
        def _():
            slot = lax.rem(i, _NBUF)
            for b in range(_NBUF):
                @pl.when(slot == b)
                def _():
                    pltpu.make_async_copy(table_hbm.at[idx_v.at[i]],
                                          rows[b], gsem[b]).wait()
                    pltpu.async_copy(
                        rows[b].at[pl.ds(0, _WROWS)],
                        out_hbm.at[pl.ds((wbase + i * _BPC) * _PAD, _WROWS)],
                        wsem[b])
        return 0

    lax.fori_loop(0, n + _D, body, 0)

    for b in range(_NBUF):
        pltpu.make_async_copy(
            rows[b].at[pl.ds(0, _WROWS)],
            out_hbm.at[pl.ds(0, _WROWS)],
            wsem[b]).wait()


def kernel(token_ids, lookup):
    bsz, seq = token_ids.shape
    num, dim = lookup.shape
    bpw = bsz // _NW
    n = bpw // _BPC

    t4 = token_ids.astype(jnp.int32).reshape(_NW, n, _BPC, seq)
    t4 = jnp.pad(t4, ((0, 0), (0, 0), (0, 0), (0, _PAD - seq)), mode="wrap")
    idx = jnp.pad(t4.reshape(_NW, n, _BPC * _PAD),
                  ((0, 0), (0, 0), (0, _CHUNK - _BPC * _PAD)), mode="wrap")

    call = functools.partial(
        pl.kernel,
        mesh=plsc.VectorSubcoreMesh(core_axis_name="c", subcore_axis_name="s"),
        out_type=jax.ShapeDtypeStruct((bsz * _PAD, dim), jnp.float32),
        scratch_types=(
            [pltpu.VMEM((n, _CHUNK), jnp.int32)]
            + [pltpu.VMEM((_CHUNK, dim), jnp.float32) for _ in range(_NBUF)]
            + [pltpu.SemaphoreType.DMA for _ in range(2 * _NBUF)]
        ),
    )(_emb_body)

    out = call(idx, lookup)
    return out.reshape(bsz, _PAD, dim)[:, :seq, :]


# speedup vs baseline: 14.5126x; 1.1655x over previous
import functools

import jax
import jax.numpy as jnp
from jax import lax
from jax.experimental import pallas as pl
from jax.experimental.pallas import tpu as pltpu
from jax.experimental.pallas import tpu_sc as plsc

_NC, _NS = 2, 16
_NW = _NC * _NS
_CHUNK = 128
_BPC = 2
_NBUF = 5
_D = 3


def _emb_body(idx_hbm, table_hbm, out_hbm, idx_v, *bufs):
    rows = bufs[:_NBUF]
    gsem = bufs[_NBUF:2 * _NBUF]
    wsem = bufs[2 * _NBUF:3 * _NBUF]

    seq = out_hbm.shape[1]
    wid = lax.axis_index("s") * _NC + lax.axis_index("c")
    n = idx_hbm.shape[1]
    wbase = wid * n * _BPC
    pltpu.sync_copy(idx_hbm.at[wid], idx_v)

    def fire_writes(c, b):
        for r in range(_BPC):
            pltpu.async_copy(rows[b].at[pl.ds(r * seq, seq)],
                             out_hbm.at[wbase + c * _BPC + r],
                             wsem[b])

    def wait_writes(b):
        for r in range(_BPC):
            pltpu.make_async_copy(rows[b].at[pl.ds(r * seq, seq)],
                                  out_hbm.at[0],
                                  wsem[b]).wait()

    def body(j, _):
        @pl.when(j < n)
        def _():
            slot = lax.rem(j, _NBUF)
            for b in range(_NBUF):
                @pl.when(slot == b)
                def _():
                    @pl.when(j >= _NBUF)
                    def _():
                        wait_writes(b)
                    pltpu.async_copy(table_hbm.at[idx_v.at[j]],
                                     rows[b], gsem[b])

        i = j - _D
        @pl.when(i >= 0)
        def _():
            slot = lax.rem(i, _NBUF)
            for b in range(_NBUF):
                @pl.when(slot == b)
                def _():
                    pltpu.make_async_copy(table_hbm.at[idx_v.at[i]],
                                          rows[b], gsem[b]).wait()
                    fire_writes(i, b)
        return 0

    lax.fori_loop(0, n + _D, body, 0)

    for b in range(_NBUF):
        wait_writes(b)


def kernel(token_ids, lookup):
    bsz, seq = token_ids.shape
    num, dim = lookup.shape
    bpw = bsz // _NW
    n = bpw // _BPC
    valid = _BPC * seq

    idx = token_ids.astype(jnp.int32).reshape(_NW, n, valid)
    idx = jnp.pad(idx, ((0, 0), (0, 0), (0, _CHUNK - valid)), mode="wrap")

    call = functools.partial(
        pl.kernel,
        mesh=plsc.VectorSubcoreMesh(core_axis_name="c", subcore_axis_name="s"),
        out_type=jax.ShapeDtypeStruct((bsz, seq, dim), jnp.float32),
        scratch_types=(
            [pltpu.VMEM((n, _CHUNK), jnp.int32)]
            + [pltpu.VMEM((_CHUNK, dim), jnp.float32) for _ in range(_NBUF)]
            + [pltpu.SemaphoreType.DMA for _ in range(2 * _NBUF)]
        ),
    )(_emb_body)

    return call(idx, lookup)


# speedup vs baseline: 15.5194x; 1.0694x over previous
import functools

import jax
import jax.numpy as jnp
from jax import lax
from jax.experimental import pallas as pl
from jax.experimental.pallas import tpu as pltpu
from jax.experimental.pallas import tpu_sc as plsc

_NC, _NS = 2, 16
_NW = _NC * _NS
_CHUNK = 128
_BPC = 2
_NBUF = 5
_D = 3


def _emb_body(idx_hbm, table_hbm, out_hbm, idx_v, *bufs):
    rows = bufs[:_NBUF]
    gsem = bufs[_NBUF:2 * _NBUF]
    wsem = bufs[2 * _NBUF:3 * _NBUF]

    seq = out_hbm.shape[1]
    valid = _BPC * seq
    wid = lax.axis_index("s") * _NC + lax.axis_index("c")
    n = idx_hbm.shape[1]
    wbase = wid * n * _BPC
    pltpu.sync_copy(idx_hbm.at[wid], idx_v)

    def fire_writes(c, b):
        for r in range(_BPC):
            pltpu.async_copy(rows[b].at[pl.ds(r * seq, seq)],
                             out_hbm.at[wbase + c * _BPC + r],
                             wsem[b])

    def wait_writes(b):
        for r in range(_BPC):
            pltpu.make_async_copy(rows[b].at[pl.ds(r * seq, seq)],
                                  out_hbm.at[0],
                                  wsem[b]).wait()

    def body(j, _):
        @pl.when(j < n)
        def _():
            slot = lax.rem(j, _NBUF)
            for b in range(_NBUF):
                @pl.when(slot == b)
                def _():
                    @pl.when(j >= _NBUF)
                    def _():
                        wait_writes(b)
                    pltpu.async_copy(
                        table_hbm.at[idx_v.at[j, pl.ds(0, valid)]],
                        rows[b], gsem[b])

        i = j - _D
        @pl.when(i >= 0)
        def _():
            slot = lax.rem(i, _NBUF)
            for b in range(_NBUF):
                @pl.when(slot == b)
                def _():
                    pltpu.make_async_copy(
                        table_hbm.at[idx_v.at[i, pl.ds(0, valid)]],
                        rows[b], gsem[b]).wait()
                    fire_writes(i, b)
        return 0

    lax.fori_loop(0, n + _D, body, 0)

    for b in range(_NBUF):
        wait_writes(b)


def kernel(token_ids, lookup):
    bsz, seq = token_ids.shape
    num, dim = lookup.shape
    bpw = bsz // _NW
    n = bpw // _BPC
    valid = _BPC * seq

    idx = token_ids.astype(jnp.int32).reshape(_NW, n, valid)
    idx = jnp.pad(idx, ((0, 0), (0, 0), (0, _CHUNK - valid)), mode="wrap")

    call = functools.partial(
        pl.kernel,
        mesh=plsc.VectorSubcoreMesh(core_axis_name="c", subcore_axis_name="s"),
        out_type=jax.ShapeDtypeStruct((bsz, seq, dim), jnp.float32),
        scratch_types=(
            [pltpu.VMEM((n, _CHUNK), jnp.int32)]
            + [pltpu.VMEM((_BPC * seq, dim), jnp.float32)
               for _ in range(_NBUF)]
            + [pltpu.SemaphoreType.DMA for _ in range(2 * _NBUF)]
        ),
    )(_emb_body)

    return call(idx, lookup)
